# baseline (speedup 1.0000x reference)
import functools

import jax
import jax.numpy as jnp
from jax import lax
from jax.experimental import pallas as pl
from jax.experimental.pallas import tpu as pltpu
from jax.experimental.pallas import tpu_sc as plsc

N_CLASSES = 1000
B, F = 1024, 26
ROWS = B * F
_INFO = plsc.get_sparse_core_info()
NC, NS = _INFO.num_cores, _INFO.num_subcores
NW = NC * NS
ROWS_PER_W = ROWS // NW
CHUNK = 64
N_CHUNKS = ROWS_PER_W // CHUNK
BUF_WORDS = CHUNK * N_CLASSES


@functools.partial(
    pl.kernel,
    mesh=plsc.VectorSubcoreMesh(core_axis_name="c", subcore_axis_name="s"),
    out_type=jax.ShapeDtypeStruct((ROWS * N_CLASSES,), jnp.int32),
    scratch_types=[
        pltpu.VMEM((ROWS_PER_W,), jnp.int32),
        pltpu.VMEM((BUF_WORDS,), jnp.int32),
    ],
    compiler_params=pltpu.CompilerParams(needs_layout_passes=False),
)
def _one_hot_sc(x_hbm, out_hbm, idx_v, buf_v):
    wid = lax.axis_index("s") * NC + lax.axis_index("c")
    base_row = wid * ROWS_PER_W
    pltpu.sync_copy(x_hbm.at[pl.ds(base_row, ROWS_PER_W)], idx_v)

    zeros16 = jnp.zeros((16,), jnp.int32)
    ones16 = jnp.ones((16,), jnp.int32)
    lane = lax.iota(jnp.int32, 16)

    def zero_body(i, carry):
        buf_v[pl.ds(i * 16, 16)] = zeros16
        return carry

    lax.fori_loop(0, BUF_WORDS // 16, zero_body, 0)

    def scatter_chunk(c, value16):
        def vec_body(v, _):
            r0 = v * 16
            idx = idx_v[pl.ds(c * CHUNK + r0, 16)]
            pos = (lane + r0) * N_CLASSES + idx
            plsc.store_scatter(buf_v, [pos], value16)
            return 0

        lax.fori_loop(0, CHUNK // 16, vec_body, 0)

    def chunk_body(c, carry):
        scatter_chunk(c, ones16)
        dst = out_hbm.at[pl.ds((base_row + c * CHUNK) * N_CLASSES, BUF_WORDS)]
        pltpu.sync_copy(buf_v, dst)
        scatter_chunk(c, zeros16)
        return carry

    lax.fori_loop(0, N_CHUNKS, chunk_body, 0)


def kernel(x):
    out = _one_hot_sc(x.reshape(ROWS))
    return out.reshape(B, F, N_CLASSES)


# speedup vs baseline: 1.8776x; 1.8776x over previous
import functools

import jax
import jax.numpy as jnp
from jax import lax
from jax.experimental import pallas as pl
from jax.experimental.pallas import tpu as pltpu
from jax.experimental.pallas import tpu_sc as plsc

N_CLASSES = 1000
B, F = 1024, 26
_INFO = plsc.get_sparse_core_info()
NC, NS = _INFO.num_cores, _INFO.num_subcores
NW = NC * NS
B_PER_W = B // NW
BB = 2
N_CHUNKS = B_PER_W // BB
ONES_PER_CHUNK = BB * F
N_VECS = (ONES_PER_CHUNK + 15) // 16


@functools.partial(
    pl.kernel,
    mesh=plsc.VectorSubcoreMesh(core_axis_name="c", subcore_axis_name="s"),
    out_type=jax.ShapeDtypeStruct((B, F, N_CLASSES), jnp.int32),
    scratch_types=[
        pltpu.VMEM((B_PER_W * F,), jnp.int32),
        pltpu.VMEM((BB, F, N_CLASSES), jnp.int32),
    ],
    compiler_params=pltpu.CompilerParams(needs_layout_passes=False),
)
def _one_hot_sc(x_hbm, out_hbm, idx_v, buf_v):
    wid = lax.axis_index("s") * NC + lax.axis_index("c")
    base_b = wid * B_PER_W
    pltpu.sync_copy(x_hbm.at[pl.ds(base_b * F, B_PER_W * F)], idx_v)

    zeros16 = jnp.zeros((16,), jnp.int32)
    ones16 = jnp.ones((16,), jnp.int32)
    lane = lax.iota(jnp.int32, 16)

    def zero_row(i, carry):
        bb = i // F
        ff = i % F
        for k in range(N_CLASSES // 16):
            buf_v[bb, ff, pl.ds(k * 16, 16)] = zeros16
        buf_v[bb, ff, pl.ds(N_CLASSES - 16, 16)] = zeros16
        return carry

    lax.fori_loop(0, BB * F, zero_row, 0)

    def scatter_chunk(c, value16):
        def vec_body(v, _):
            r0 = v * 16
            r = lane + r0
            mask = r < ONES_PER_CHUNK
            idx = idx_v[pl.ds(c * ONES_PER_CHUNK + r0, 16)]
            bb = r // F
            ff = r % F
            plsc.store_scatter(buf_v, [bb, ff, idx], value16, mask=mask)
            return 0

        lax.fori_loop(0, N_VECS, vec_body, 0)

    def chunk_body(c, carry):
        scatter_chunk(c, ones16)
        pltpu.sync_copy(buf_v, out_hbm.at[pl.ds(base_b + c * BB, BB)])
        scatter_chunk(c, zeros16)
        return carry

    lax.fori_loop(0, N_CHUNKS, chunk_body, 0)


def kernel(x):
    return _one_hot_sc(x.reshape(B * F))
